# baseline (speedup 1.0000x reference)
import functools

import jax
import jax.numpy as jnp
from jax import lax
from jax.experimental import pallas as pl
from jax.experimental.pallas import tpu as pltpu
from jax.experimental.pallas import tpu_sc as plsc

N_PEAKS_ = 64
DIM_ = 128


def _cond_body(x_ref, s_ref, b_ref, o_ref):
    s = s_ref[...][:, :, None]
    b = b_ref[...][:, :, None]
    o_ref[...] = x_ref[...] * s + b


_L = 16


def _sc_body(nrows_w, dim, t, x_hbm, ws16_hbm, wb16_hbm, out_hbm, s16_v, b16_v, row_v):
    nc = 2
    wid = lax.axis_index("s") * nc + lax.axis_index("c")
    row_w = dim * t
    wrow_w = dim * _L

    def do_row(k, carry):
        row = wid * nrows_w + k
        base = row * row_w
        pltpu.sync_copy(ws16_hbm.at[pl.ds(k * wrow_w, wrow_w)], s16_v)
        pltpu.sync_copy(wb16_hbm.at[pl.ds(k * wrow_w, wrow_w)], b16_v)
        pltpu.sync_copy(x_hbm.at[pl.ds(base, row_w)], row_v)

        def do_d(d, carry2):
            s = s16_v[pl.ds(d * _L, _L)]
            b = b16_v[pl.ds(d * _L, _L)]
            for tt in range(t // _L):
                sl = pl.ds(d * t + tt * _L, _L)
                row_v[sl] = row_v[sl] * s + b
            return carry2

        lax.fori_loop(0, dim, do_d, 0)
        pltpu.sync_copy(row_v, out_hbm.at[pl.ds(base, row_w)])
        return carry

    lax.fori_loop(0, nrows_w, do_row, 0)


def _sc_kernel(x, W_scale, W_bias):
    rows, dim, t = x.shape
    nw = 32
    nrows_w = rows // nw
    assert nrows_w == N_PEAKS_
    xf = x.reshape(rows * dim * t)
    ws16 = jnp.repeat(W_scale.reshape(N_PEAKS_, dim, 1), _L, axis=2).reshape(-1)
    wb16 = jnp.repeat(W_bias.reshape(N_PEAKS_, dim, 1), _L, axis=2).reshape(-1)
    mesh = plsc.VectorSubcoreMesh(core_axis_name="c", subcore_axis_name="s")
    f = pl.kernel(
        functools.partial(_sc_body, nrows_w, dim, t),
        out_type=jax.ShapeDtypeStruct((rows * dim * t,), x.dtype),
        mesh=mesh,
        scratch_types=[
            pltpu.VMEM((dim * _L,), jnp.float32),
            pltpu.VMEM((dim * _L,), jnp.float32),
            pltpu.VMEM((dim * t,), jnp.float32),
        ],
    )
    out = f(xf, ws16, wb16)
    return out.reshape(x.shape)


def kernel(x, queries, W_scale, W_bias):
    del queries
    return _sc_kernel(x, W_scale, W_bias)
    rows, dim, t = x.shape
    R = 64
    grid = (rows // R,)

    out = pl.pallas_call(
        _cond_body,
        grid=grid,
        in_specs=[
            pl.BlockSpec((R, dim, t), lambda i: (i, 0, 0)),
            pl.BlockSpec((N_PEAKS_, dim), lambda i: (0, 0)),
            pl.BlockSpec((N_PEAKS_, dim), lambda i: (0, 0)),
        ],
        out_specs=pl.BlockSpec((R, dim, t), lambda i: (i, 0, 0)),
        out_shape=jax.ShapeDtypeStruct(x.shape, x.dtype),
        compiler_params=pltpu.CompilerParams(
            dimension_semantics=("parallel",),
        ),
    )(x, W_scale, W_bias)
    return out
